# Initial kernel scaffold; baseline (speedup 1.0000x reference)
import jax
import jax.numpy as jnp
from jax import lax
from jax.experimental import pallas as pl
from jax.experimental.pallas import tpu as pltpu

_K = 64
_ROWS_PER_BLOCK = 8


def _topk_mask_kernel(z_ref, out_ref):
    z = z_ref[...]
    b = lax.bitcast_convert_type(z, jnp.int32)
    v = jnp.where(b < 0, b ^ 0x7FFFFFFF, b)
    lo0 = jnp.min(v, axis=1, keepdims=True)
    hi0 = jnp.max(v, axis=1, keepdims=True)

    def cond(state):
        lo, hi = state
        return jnp.any(lo < hi)

    def body(state):
        lo, hi = state
        d = hi - lo
        mid = lo + lax.shift_right_logical(d, 1) + (d & 1)
        cnt = jnp.sum((v >= mid).astype(jnp.int32), axis=1, keepdims=True)
        ge = cnt >= _K
        eq = cnt == _K
        new_lo = jnp.where(ge, mid, lo)
        new_hi = jnp.where(eq, mid, jnp.where(ge, hi, mid - 1))
        return new_lo, new_hi

    thr, _ = lax.while_loop(cond, body, (lo0, hi0))
    out_ref[...] = jnp.where(v >= thr, z, 0.0)


def kernel(z):
    rows, cols = z.shape
    return pl.pallas_call(
        _topk_mask_kernel,
        grid=(rows // _ROWS_PER_BLOCK,),
        in_specs=[pl.BlockSpec((_ROWS_PER_BLOCK, cols), lambda i: (i, 0))],
        out_specs=pl.BlockSpec((_ROWS_PER_BLOCK, cols), lambda i: (i, 0)),
        out_shape=jax.ShapeDtypeStruct((rows, cols), z.dtype),
        compiler_params=pltpu.CompilerParams(
            dimension_semantics=("arbitrary",),
        ),
    )(z)


# speedup vs baseline: 6.1246x; 6.1246x over previous
import jax
import jax.numpy as jnp
from jax import lax
from jax.experimental import pallas as pl
from jax.experimental.pallas import tpu as pltpu

_K = 64
_ROWS_PER_BLOCK = 8


def _topk_mask_kernel(z_ref, out_ref):
    z = z_ref[...]
    b = lax.bitcast_convert_type(z, jnp.int32)
    v = jnp.where(b < 0, b ^ 0x7FFFFFFF, b)
    lo0 = jnp.min(v, axis=1, keepdims=True)
    hi0 = jnp.max(v, axis=1, keepdims=True)

    cnt0 = jnp.full_like(lo0, v.shape[1])

    def cond(state):
        lo, hi, _ = state
        return jnp.any(lo < hi)

    def body(state):
        lo, hi, cnt = state
        d = hi - lo
        mid = lo + lax.shift_right_logical(d, 1) + (d & 1)
        c = jnp.sum((v >= mid).astype(jnp.int32), axis=1, keepdims=True)
        ge = c >= _K
        eq = c == _K
        new_lo = jnp.where(ge, mid, lo)
        new_hi = jnp.where(eq, mid, jnp.where(ge, hi, mid - 1))
        new_cnt = jnp.where(ge, c, cnt)
        return new_lo, new_hi, new_cnt

    thr, _, cntf = lax.while_loop(cond, body, (lo0, hi0, cnt0))

    col = lax.broadcasted_iota(jnp.int32, v.shape, 1)
    eqm = v == thr
    eqc = jnp.sum(eqm.astype(jnp.int32), axis=1, keepdims=True)
    need = _K - (cntf - eqc)
    last = v.shape[1] - 1
    clo0 = jnp.where(cntf > _K, 0, last)
    chi0 = jnp.full_like(clo0, last)

    def tcond(state):
        clo, chi = state
        return jnp.any(clo < chi)

    def tbody(state):
        clo, chi = state
        mid = clo + lax.shift_right_logical(chi - clo, 1)
        g = jnp.sum((eqm & (col <= mid)).astype(jnp.int32), axis=1,
                    keepdims=True)
        ok = g >= need
        return jnp.where(ok, clo, mid + 1), jnp.where(ok, mid, chi)

    cstar, _ = lax.while_loop(tcond, tbody, (clo0, chi0))

    keep = (v > thr) | (eqm & (col <= cstar))
    out_ref[...] = jnp.where(keep, z, 0.0)


def kernel(z):
    rows, cols = z.shape
    return pl.pallas_call(
        _topk_mask_kernel,
        grid=(rows // _ROWS_PER_BLOCK,),
        in_specs=[pl.BlockSpec((_ROWS_PER_BLOCK, cols), lambda i: (i, 0))],
        out_specs=pl.BlockSpec((_ROWS_PER_BLOCK, cols), lambda i: (i, 0)),
        out_shape=jax.ShapeDtypeStruct((rows, cols), z.dtype),
        compiler_params=pltpu.CompilerParams(
            dimension_semantics=("arbitrary",),
        ),
    )(z)


# speedup vs baseline: 10.3705x; 1.6933x over previous
import jax
import jax.numpy as jnp
from jax import lax
from jax.experimental import pallas as pl
from jax.experimental.pallas import tpu as pltpu

_K = 64
_ROWS_PER_BLOCK = 8


def _count_ge(v, c):
    return jnp.sum((v >= c).astype(jnp.int32), axis=1, keepdims=True)


def _topk_mask_kernel(z_ref, out_ref):
    z = z_ref[...]
    b = lax.bitcast_convert_type(z, jnp.int32)
    v = jnp.where(b < 0, b ^ 0x7FFFFFFF, b)
    lo0 = jnp.min(v, axis=1, keepdims=True)
    hi0 = jnp.max(v, axis=1, keepdims=True)
    c_lo0 = jnp.full_like(lo0, v.shape[1])
    c_hi0 = jnp.ones_like(lo0)

    def cond(state):
        lo, _, hi, _, _ = state
        return jnp.any(lo < hi)

    def body(state):
        lo, c_lo, hi, c_hi, k = state
        d = hi - lo
        d_f = d.astype(jnp.float32) + jnp.where(d < 0, 4294967296.0, 0.0)
        frac = (c_lo - _K).astype(jnp.float32) / jnp.maximum(
            c_lo - c_hi, 1).astype(jnp.float32)
        frac = jnp.clip(frac, 0.0, 1.0)
        off1 = (d_f * frac * 0.5).astype(jnp.int32)
        sec = jnp.clip(lo + off1 + off1, lo + 1, hi)
        bis = lo + lax.shift_right_logical(d, 1) + (d & 1)
        mid = jnp.where((k & 1) == 0, sec, bis)
        c = _count_ge(v, mid)
        ge = c >= _K
        eq = c == _K
        new_lo = jnp.where(ge, mid, lo)
        new_clo = jnp.where(ge, c, c_lo)
        new_hi = jnp.where(eq, mid, jnp.where(ge, hi, mid - 1))
        new_chi = jnp.where(ge, c_hi, c)
        return new_lo, new_clo, new_hi, new_chi, k + 1

    thr, cntf, _, _, _ = lax.while_loop(
        cond, body, (lo0, c_lo0, hi0, c_hi0, jnp.int32(0)))

    surplus = jnp.any(cntf > _K)

    @pl.when(jnp.logical_not(surplus))
    def _simple():
        out_ref[...] = jnp.where(v >= thr, z, 0.0)

    @pl.when(surplus)
    def _ties():
        col = lax.broadcasted_iota(jnp.int32, v.shape, 1)
        eqm = v == thr
        eqc = jnp.sum(eqm.astype(jnp.int32), axis=1, keepdims=True)
        need = _K - (cntf - eqc)
        last = v.shape[1] - 1
        clo0 = jnp.where(cntf > _K, 0, last)
        chi0 = jnp.full_like(clo0, last)

        def tcond(state):
            clo, chi = state
            return jnp.any(clo < chi)

        def tbody(state):
            clo, chi = state
            mid = clo + lax.shift_right_logical(chi - clo, 1)
            g = jnp.sum((eqm & (col <= mid)).astype(jnp.int32), axis=1,
                        keepdims=True)
            ok = g >= need
            return jnp.where(ok, clo, mid + 1), jnp.where(ok, mid, chi)

        cstar, _ = lax.while_loop(tcond, tbody, (clo0, chi0))
        keep = (v > thr) | (eqm & (col <= cstar))
        out_ref[...] = jnp.where(keep, z, 0.0)


def kernel(z):
    rows, cols = z.shape
    return pl.pallas_call(
        _topk_mask_kernel,
        grid=(rows // _ROWS_PER_BLOCK,),
        in_specs=[pl.BlockSpec((_ROWS_PER_BLOCK, cols), lambda i: (i, 0))],
        out_specs=pl.BlockSpec((_ROWS_PER_BLOCK, cols), lambda i: (i, 0)),
        out_shape=jax.ShapeDtypeStruct((rows, cols), z.dtype),
        compiler_params=pltpu.CompilerParams(
            dimension_semantics=("arbitrary",),
        ),
    )(z)
